# Initial kernel scaffold; baseline (speedup 1.0000x reference)
import jax
import jax.numpy as jnp
from jax.experimental import pallas as pl


def kernel(x, edge_index, W1, b1, gamma, beta, W2, b2):
    raise NotImplementedError("write your pallas kernel here")


# speedup vs baseline: 15.5241x; 15.5241x over previous
import functools

import jax
import jax.numpy as jnp
from jax import lax
from jax.experimental import pallas as pl
from jax.experimental.pallas import tpu as pltpu
from jax.experimental.pallas import tpu_sc as plsc

N = 10000
E = 320000
IN_CH = 128
HIDDEN = 64
OUT_CH = 32
BN_EPS = 1e-5

NC = 2
NS = 16
NW = NC * NS

EB = 128
EP = 327680
EROWS = EP // EB
RW = EROWS // NW
NPAD = 10112
RT = NPAD // NS

_f32 = jnp.float32


def _sc_mesh():
    return plsc.VectorSubcoreMesh(
        core_axis_name="c", subcore_axis_name="s", num_cores=NC, num_subcores=NS
    )


_SC_PARAMS = pltpu.CompilerParams(use_tc_tiling_on_sc=False)


def _deg_body(dst_hbm, ones_hbm, zeros_hbm, out_hbm, acc_sh, dstbuf, onesbuf, cbuf, sem):
    c = lax.axis_index("c")
    s = lax.axis_index("s")
    wid = s * NC + c
    r0 = s * RT
    pltpu.sync_copy(zeros_hbm.at[pl.ds(r0, RT)], acc_sh.at[pl.ds(r0, RT)])
    pltpu.sync_copy(ones_hbm, onesbuf)
    pltpu.sync_copy(dst_hbm.at[pl.ds(wid * RW, RW)], dstbuf)
    plsc.subcore_barrier()

    def step(j, _):
        pltpu.sync_copy(onesbuf, acc_sh.at[dstbuf.at[j]], add=True)
        return 0

    lax.fori_loop(0, RW, step, 0)
    plsc.subcore_barrier()
    pltpu.sync_copy(acc_sh.at[pl.ds(r0, RT)], cbuf)
    pltpu.sync_copy(cbuf, out_hbm.at[pl.ds(c * NPAD + r0, RT)])


def _edge_body(g_hbm, src_hbm, dst_hbm, zeros_hbm, out_hbm,
               acc_sh, srcbuf, dstbuf, rows, cbuf, sem, *, F):
    c = lax.axis_index("c")
    s = lax.axis_index("s")
    wid = s * NC + c
    r0 = s * RT
    pltpu.sync_copy(zeros_hbm.at[pl.ds(r0, RT)], acc_sh.at[pl.ds(r0, RT)])
    pltpu.sync_copy(src_hbm.at[pl.ds(wid * RW, RW)], srcbuf)
    pltpu.sync_copy(dst_hbm.at[pl.ds(wid * RW, RW)], dstbuf)
    plsc.subcore_barrier()

    def step(j, _):
        pltpu.async_copy(g_hbm.at[srcbuf.at[j]], rows, sem).wait()
        pltpu.sync_copy(rows, acc_sh.at[dstbuf.at[j]], add=True)
        return 0

    lax.fori_loop(0, RW, step, 0)
    plsc.subcore_barrier()
    pltpu.sync_copy(acc_sh.at[pl.ds(r0, RT)], cbuf)
    pltpu.sync_copy(cbuf, out_hbm.at[pl.ds(c * NPAD + r0, RT)])


def _sc_deg(dst_rows, ones, zeros8):
    return pl.kernel(
        _deg_body,
        out_type=jax.ShapeDtypeStruct((NC * NPAD, 8), _f32),
        mesh=_sc_mesh(),
        scratch_types=[
            pltpu.VMEM_SHARED((NPAD, 8), _f32),
            pltpu.VMEM((RW, EB), jnp.int32),
            pltpu.VMEM((EB, 8), _f32),
            pltpu.VMEM((RT, 8), _f32),
            pltpu.SemaphoreType.DMA,
        ],
        compiler_params=_SC_PARAMS,
    )(dst_rows, ones, zeros8)


def _sc_edge(g, src_rows, dst_rows, zerosF, F):
    return pl.kernel(
        functools.partial(_edge_body, F=F),
        out_type=jax.ShapeDtypeStruct((NC * NPAD, F), _f32),
        mesh=_sc_mesh(),
        scratch_types=[
            pltpu.VMEM_SHARED((NPAD, F), _f32),
            pltpu.VMEM((RW, EB), jnp.int32),
            pltpu.VMEM((RW, EB), jnp.int32),
            pltpu.VMEM((EB, F), _f32),
            pltpu.VMEM((RT, F), _f32),
            pltpu.SemaphoreType.DMA,
        ],
        compiler_params=_SC_PARAMS,
    )(g, src_rows, dst_rows, zerosF)


def _prep1_body(xp_ref, w1_ref, degp_ref, g1_ref, dinv_ref):
    deg = degp_ref[0:NPAD, 0] + degp_ref[NPAD:2 * NPAD, 0] + 1.0
    dinv = lax.rsqrt(deg)
    h = jnp.dot(xp_ref[...], w1_ref[...], preferred_element_type=_f32)
    g1_ref[...] = h * dinv[:, None]
    dinv_ref[...] = dinv


def _mid_body(a1p_ref, g1_ref, dinv_ref, b1_ref, gamma_ref, beta_ref, w2_ref, g2_ref):
    dinv = dinv_ref[...]
    a1 = a1p_ref[0:NPAD, :] + a1p_ref[NPAD:2 * NPAD, :]
    pre = dinv[:, None] * (a1 + g1_ref[...]) + b1_ref[...][None, :]
    live = pre[0:N, :]
    mean = jnp.sum(live, axis=0) * (1.0 / N)
    var = jnp.sum((live - mean[None, :]) ** 2, axis=0) * (1.0 / N)
    hb = (pre - mean[None, :]) * lax.rsqrt(var + BN_EPS)[None, :]
    hb = hb * gamma_ref[...][None, :] + beta_ref[...][None, :]
    r = jnp.maximum(hb, 0.0)
    h2 = jnp.dot(r, w2_ref[...], preferred_element_type=_f32)
    g2_ref[...] = h2 * dinv[:, None]


def _final_body(a2p_ref, g2_ref, dinv_ref, b2_ref, z_ref):
    a2 = a2p_ref[0:NPAD, :] + a2p_ref[NPAD:2 * NPAD, :]
    z = dinv_ref[...][:, None] * (a2 + g2_ref[...]) + b2_ref[...][None, :]
    z_ref[...] = z[0:N, :]


def kernel(x, edge_index, W1, b1, gamma, beta, W2, b2):
    src = edge_index[0].astype(jnp.int32)
    dst = edge_index[1].astype(jnp.int32)
    pad = jnp.full((EP - E,), N, jnp.int32)
    src_rows = jnp.concatenate([src, pad]).reshape(EROWS, EB)
    dst_rows = jnp.concatenate([dst, pad]).reshape(EROWS, EB)
    xp = jnp.zeros((NPAD, IN_CH), _f32).at[0:N, :].set(x)
    ones = jnp.ones((EB, 8), _f32)
    zeros8 = jnp.zeros((NPAD, 8), _f32)
    zeros64 = jnp.zeros((NPAD, HIDDEN), _f32)
    zeros32 = jnp.zeros((NPAD, OUT_CH), _f32)

    deg_parts = _sc_deg(dst_rows, ones, zeros8)

    g1, dinv = pl.pallas_call(
        _prep1_body,
        out_shape=(
            jax.ShapeDtypeStruct((NPAD, HIDDEN), _f32),
            jax.ShapeDtypeStruct((NPAD,), _f32),
        ),
    )(xp, W1, deg_parts)

    a1_parts = _sc_edge(g1, src_rows, dst_rows, zeros64, HIDDEN)

    g2 = pl.pallas_call(
        _mid_body,
        out_shape=jax.ShapeDtypeStruct((NPAD, OUT_CH), _f32),
    )(a1_parts, g1, dinv, b1, gamma, beta, W2)

    a2_parts = _sc_edge(g2, src_rows, dst_rows, zeros32, OUT_CH)

    z = pl.pallas_call(
        _final_body,
        out_shape=jax.ShapeDtypeStruct((N, OUT_CH), _f32),
    )(a2_parts, g2, dinv, b2)
    return z


# speedup vs baseline: 17.9276x; 1.1548x over previous
import jax
import jax.numpy as jnp
from jax import lax
from jax.experimental import pallas as pl
from jax.experimental.pallas import tpu as pltpu
from jax.experimental.pallas import tpu_sc as plsc

N = 10000
E = 320000
IN_CH = 128
HIDDEN = 64
OUT_CH = 32
BN_EPS = 1e-5

NC = 2
NS = 16
NW = NC * NS

EB = 128
EP = 327680
EROWS = EP // EB
RW = EROWS // NW
NPAD = 10112
RT = NPAD // NS

_f32 = jnp.float32


def _sc_mesh():
    return plsc.VectorSubcoreMesh(
        core_axis_name="c", subcore_axis_name="s", num_cores=NC, num_subcores=NS
    )


_SC_PARAMS = pltpu.CompilerParams(use_tc_tiling_on_sc=False)


def _deg_body(dst_hbm, ones_hbm, zeros_hbm, out_hbm, acc_sh, dstbuf, onesbuf, sem):
    c = lax.axis_index("c")
    s = lax.axis_index("s")
    wid = s * NC + c
    r0 = s * RT
    pltpu.sync_copy(zeros_hbm.at[pl.ds(r0, RT)], acc_sh.at[pl.ds(r0, RT)])
    pltpu.sync_copy(ones_hbm, onesbuf)
    pltpu.sync_copy(dst_hbm.at[pl.ds(wid * RW, RW)], dstbuf)
    plsc.subcore_barrier()

    def step(j, _):
        pltpu.sync_copy(onesbuf, acc_sh.at[dstbuf.at[j]], add=True)
        return 0

    lax.fori_loop(0, RW, step, 0)
    plsc.subcore_barrier()
    pltpu.sync_copy(acc_sh.at[pl.ds(r0, RT)], out_hbm.at[pl.ds(c * NPAD + r0, RT)])


NB = 4
NBLK2 = RW // (2 * NB)


def _edge_body(g_hbm, src_hbm, dst_hbm, zeros_hbm, out_hbm,
               acc_sh, srcbuf, dstbuf, *bufs_and_sems):
    rows = bufs_and_sems[0:2 * NB]
    gsem = bufs_and_sems[2 * NB:3 * NB]
    ssem = bufs_and_sems[3 * NB:5 * NB]
    c = lax.axis_index("c")
    s = lax.axis_index("s")
    wid = s * NC + c
    r0 = s * RT
    pltpu.sync_copy(zeros_hbm.at[pl.ds(r0, RT)], acc_sh.at[pl.ds(r0, RT)])
    pltpu.sync_copy(src_hbm.at[pl.ds(wid * RW, RW)], srcbuf)
    pltpu.sync_copy(dst_hbm.at[pl.ds(wid * RW, RW)], dstbuf)
    plsc.subcore_barrier()

    def body(i, _):
        for half in range(2):
            j0 = (2 * i + half) * NB
            hb = rows[half * NB:(half + 1) * NB]
            hs = ssem[half * NB:(half + 1) * NB]

            @pl.when(i > 0)
            def _wait_prev():
                for b in range(NB):
                    pltpu.make_async_copy(hb[b], acc_sh.at[dstbuf.at[0]], hs[b]).wait()

            gd = [pltpu.async_copy(g_hbm.at[srcbuf.at[j0 + b]], hb[b], gsem[b])
                  for b in range(NB)]
            for b in range(NB):
                gd[b].wait()
                pltpu.async_copy(hb[b], acc_sh.at[dstbuf.at[j0 + b]], hs[b], add=True)
        return 0

    lax.fori_loop(0, NBLK2, body, 0)
    for b in range(2 * NB):
        pltpu.make_async_copy(rows[b], acc_sh.at[dstbuf.at[0]], ssem[b]).wait()
    plsc.subcore_barrier()
    pltpu.sync_copy(acc_sh.at[pl.ds(r0, RT)], out_hbm.at[pl.ds(c * NPAD + r0, RT)])


def _sc_deg(dst_rows, ones, zeros8):
    return pl.kernel(
        _deg_body,
        out_type=jax.ShapeDtypeStruct((NC * NPAD, 8), _f32),
        mesh=_sc_mesh(),
        scratch_types=[
            pltpu.VMEM_SHARED((NPAD, 8), _f32),
            pltpu.VMEM((RW, EB), jnp.int32),
            pltpu.VMEM((EB, 8), _f32),
            pltpu.SemaphoreType.DMA,
        ],
        compiler_params=_SC_PARAMS,
    )(dst_rows, ones, zeros8)


def _sc_edge(g, src_rows, dst_rows, zerosF, F):
    return pl.kernel(
        _edge_body,
        out_type=jax.ShapeDtypeStruct((NC * NPAD, F), _f32),
        mesh=_sc_mesh(),
        scratch_types=(
            [
                pltpu.VMEM_SHARED((NPAD, F), _f32),
                pltpu.VMEM((RW, EB), jnp.int32),
                pltpu.VMEM((RW, EB), jnp.int32),
            ]
            + [pltpu.VMEM((EB, F), _f32)] * (2 * NB)
            + [pltpu.SemaphoreType.DMA] * (3 * NB)
        ),
        compiler_params=_SC_PARAMS,
    )(g, src_rows, dst_rows, zerosF)


def _prep1_body(xp_ref, w1_ref, degp_ref, g1_ref, dinv_ref):
    deg = degp_ref[0:NPAD, 0] + degp_ref[NPAD:2 * NPAD, 0] + 1.0
    dinv = lax.rsqrt(deg)
    h = jnp.dot(xp_ref[...], w1_ref[...], preferred_element_type=_f32)
    g1_ref[...] = h * dinv[:, None]
    dinv_ref[...] = dinv


def _mid_body(a1p_ref, g1_ref, dinv_ref, b1_ref, gamma_ref, beta_ref, w2_ref, g2_ref):
    dinv = dinv_ref[...]
    a1 = a1p_ref[0:NPAD, :] + a1p_ref[NPAD:2 * NPAD, :]
    pre = dinv[:, None] * (a1 + g1_ref[...]) + b1_ref[...][None, :]
    live = pre[0:N, :]
    mean = jnp.sum(live, axis=0) * (1.0 / N)
    var = jnp.sum((live - mean[None, :]) ** 2, axis=0) * (1.0 / N)
    hb = (pre - mean[None, :]) * lax.rsqrt(var + BN_EPS)[None, :]
    hb = hb * gamma_ref[...][None, :] + beta_ref[...][None, :]
    r = jnp.maximum(hb, 0.0)
    h2 = jnp.dot(r, w2_ref[...], preferred_element_type=_f32)
    g2_ref[...] = h2 * dinv[:, None]


def _final_body(a2p_ref, g2_ref, dinv_ref, b2_ref, z_ref):
    a2 = a2p_ref[0:NPAD, :] + a2p_ref[NPAD:2 * NPAD, :]
    z = dinv_ref[...][:, None] * (a2 + g2_ref[...]) + b2_ref[...][None, :]
    z_ref[...] = z[0:N, :]


def kernel(x, edge_index, W1, b1, gamma, beta, W2, b2):
    src = edge_index[0].astype(jnp.int32)
    dst = edge_index[1].astype(jnp.int32)
    pad = jnp.full((EP - E,), N, jnp.int32)
    src_rows = jnp.concatenate([src, pad]).reshape(EROWS, EB)
    dst_rows = jnp.concatenate([dst, pad]).reshape(EROWS, EB)
    xp = jnp.zeros((NPAD, IN_CH), _f32).at[0:N, :].set(x)
    ones = jnp.ones((EB, 8), _f32)
    zeros8 = jnp.zeros((NPAD, 8), _f32)
    zeros64 = jnp.zeros((NPAD, HIDDEN), _f32)
    zeros32 = jnp.zeros((NPAD, OUT_CH), _f32)

    deg_parts = _sc_deg(dst_rows, ones, zeros8)

    g1, dinv = pl.pallas_call(
        _prep1_body,
        out_shape=(
            jax.ShapeDtypeStruct((NPAD, HIDDEN), _f32),
            jax.ShapeDtypeStruct((NPAD,), _f32),
        ),
    )(xp, W1, deg_parts)

    a1_parts = _sc_edge(g1, src_rows, dst_rows, zeros64, HIDDEN)

    g2 = pl.pallas_call(
        _mid_body,
        out_shape=jax.ShapeDtypeStruct((NPAD, OUT_CH), _f32),
    )(a1_parts, g1, dinv, b1, gamma, beta, W2)

    a2_parts = _sc_edge(g2, src_rows, dst_rows, zeros32, OUT_CH)

    z = pl.pallas_call(
        _final_body,
        out_shape=jax.ShapeDtypeStruct((N, OUT_CH), _f32),
    )(a2_parts, g2, dinv, b2)
    return z


# speedup vs baseline: 44.7386x; 2.4955x over previous
import jax
import jax.numpy as jnp
from jax import lax
from jax.experimental import pallas as pl
from jax.experimental.pallas import tpu as pltpu
from jax.experimental.pallas import tpu_sc as plsc

N = 10000
E = 320000
IN_CH = 128
HIDDEN = 64
OUT_CH = 32
BN_EPS = 1e-5

NC = 2
NS = 16
NW = NC * NS

EB = 128
EP = 327680
EROWS = EP // EB
RW = EROWS // NW
NPAD = 10112
RT = NPAD // NS

_f32 = jnp.float32


def _sc_mesh():
    return plsc.VectorSubcoreMesh(
        core_axis_name="c", subcore_axis_name="s", num_cores=NC, num_subcores=NS
    )


_SC_PARAMS = pltpu.CompilerParams(use_tc_tiling_on_sc=False)


def _deg_body(dst_hbm, ones_hbm, zeros_hbm, out_hbm, acc_sh, dstbuf, onesbuf, sem):
    c = lax.axis_index("c")
    s = lax.axis_index("s")
    wid = s * NC + c
    r0 = s * RT
    pltpu.sync_copy(zeros_hbm.at[pl.ds(r0, RT)], acc_sh.at[pl.ds(r0, RT)])
    pltpu.sync_copy(ones_hbm, onesbuf)
    pltpu.sync_copy(dst_hbm.at[pl.ds(wid * RW, RW)], dstbuf)
    plsc.subcore_barrier()

    def step(j, _):
        pltpu.sync_copy(onesbuf, acc_sh.at[dstbuf.at[j]], add=True)
        return 0

    lax.fori_loop(0, RW, step, 0)
    plsc.subcore_barrier()
    pltpu.sync_copy(acc_sh.at[pl.ds(r0, RT)], out_hbm.at[pl.ds(c * NPAD + r0, RT)])


NB = 4
NBLK2 = RW // (2 * NB)


def _edge_body(g_hbm, src_hbm, dst_hbm, zeros_hbm, out_hbm,
               acc_sh, srcbuf, dstbuf, *bufs_and_sems):
    rows = bufs_and_sems[0:2 * NB]
    gsem = bufs_and_sems[2 * NB:3 * NB]
    ssem = bufs_and_sems[3 * NB:5 * NB]
    c = lax.axis_index("c")
    s = lax.axis_index("s")
    wid = s * NC + c
    r0 = s * RT
    pltpu.sync_copy(zeros_hbm.at[pl.ds(r0, RT)], acc_sh.at[pl.ds(r0, RT)])
    pltpu.sync_copy(src_hbm.at[pl.ds(wid * RW, RW)], srcbuf)
    pltpu.sync_copy(dst_hbm.at[pl.ds(wid * RW, RW)], dstbuf)
    plsc.subcore_barrier()

    def body(i, _):
        for half in range(2):
            j0 = (2 * i + half) * NB
            hb = rows[half * NB:(half + 1) * NB]
            hs = ssem[half * NB:(half + 1) * NB]

            @pl.when(i > 0)
            def _wait_prev():
                for b in range(NB):
                    pltpu.make_async_copy(hb[b], acc_sh.at[dstbuf.at[0]], hs[b]).wait()

            gd = [pltpu.async_copy(g_hbm.at[srcbuf.at[j0 + b]], hb[b], gsem[b])
                  for b in range(NB)]
            for b in range(NB):
                gd[b].wait()
                pltpu.async_copy(hb[b], acc_sh.at[dstbuf.at[j0 + b]], hs[b], add=True)
        return 0

    lax.fori_loop(0, NBLK2, body, 0)
    for b in range(2 * NB):
        pltpu.make_async_copy(rows[b], acc_sh.at[dstbuf.at[0]], ssem[b]).wait()
    plsc.subcore_barrier()
    pltpu.sync_copy(acc_sh.at[pl.ds(r0, RT)], out_hbm.at[pl.ds(c * NPAD + r0, RT)])


def _sc_deg(dst_rows, ones, zeros8):
    return pl.kernel(
        _deg_body,
        out_type=jax.ShapeDtypeStruct((NC * NPAD, 8), _f32),
        mesh=_sc_mesh(),
        scratch_types=[
            pltpu.VMEM_SHARED((NPAD, 8), _f32),
            pltpu.VMEM((RW, EB), jnp.int32),
            pltpu.VMEM((EB, 8), _f32),
            pltpu.SemaphoreType.DMA,
        ],
        compiler_params=_SC_PARAMS,
    )(dst_rows, ones, zeros8)


def _sc_edge(g, src_rows, dst_rows, zerosF, F):
    return pl.kernel(
        _edge_body,
        out_type=jax.ShapeDtypeStruct((NC * NPAD, F), _f32),
        mesh=_sc_mesh(),
        scratch_types=(
            [
                pltpu.VMEM_SHARED((NPAD, F), _f32),
                pltpu.VMEM((RW, EB), jnp.int32),
                pltpu.VMEM((RW, EB), jnp.int32),
            ]
            + [pltpu.VMEM((EB, F), _f32)] * (2 * NB)
            + [pltpu.SemaphoreType.DMA] * (3 * NB)
        ),
        compiler_params=_SC_PARAMS,
    )(g, src_rows, dst_rows, zerosF)


def _prep1_body(xp_ref, w1_ref, degp_ref, g1_ref, dinv_ref):
    deg = degp_ref[0:NPAD, 0] + degp_ref[NPAD:2 * NPAD, 0] + 1.0
    dinv = lax.rsqrt(deg)
    h = jnp.dot(xp_ref[...], w1_ref[...], preferred_element_type=_f32)
    g1_ref[...] = h * dinv[:, None]
    dinv_ref[...] = dinv


def _mid_body(a1p_ref, g1_ref, dinv_ref, b1_ref, gamma_ref, beta_ref, w2_ref, g2_ref):
    dinv = dinv_ref[...]
    a1 = a1p_ref[0:NPAD, :] + a1p_ref[NPAD:2 * NPAD, :]
    pre = dinv[:, None] * (a1 + g1_ref[...]) + b1_ref[...][None, :]
    live = pre[0:N, :]
    mean = jnp.sum(live, axis=0) * (1.0 / N)
    var = jnp.sum((live - mean[None, :]) ** 2, axis=0) * (1.0 / N)
    hb = (pre - mean[None, :]) * lax.rsqrt(var + BN_EPS)[None, :]
    hb = hb * gamma_ref[...][None, :] + beta_ref[...][None, :]
    r = jnp.maximum(hb, 0.0)
    h2 = jnp.dot(r, w2_ref[...], preferred_element_type=_f32)
    g2_ref[...] = h2 * dinv[:, None]


def _final_body(a2p_ref, g2_ref, dinv_ref, b2_ref, z_ref):
    a2 = a2p_ref[0:NPAD, :] + a2p_ref[NPAD:2 * NPAD, :]
    z = dinv_ref[...][:, None] * (a2 + g2_ref[...]) + b2_ref[...][None, :]
    z_ref[...] = z[0:N, :]


def kernel(x, edge_index, W1, b1, gamma, beta, W2, b2):
    src = edge_index[0].astype(jnp.int32)
    dst = edge_index[1].astype(jnp.int32)
    pad = N + jnp.arange(EP - E, dtype=jnp.int32) % (NPAD - N)
    src_rows = jnp.concatenate([src, pad]).reshape(EROWS, EB)
    dst_rows = jnp.concatenate([dst, pad]).reshape(EROWS, EB)
    xp = jnp.zeros((NPAD, IN_CH), _f32).at[0:N, :].set(x)
    ones = jnp.ones((EB, 8), _f32)
    zeros8 = jnp.zeros((NPAD, 8), _f32)
    zeros64 = jnp.zeros((NPAD, HIDDEN), _f32)
    zeros32 = jnp.zeros((NPAD, OUT_CH), _f32)

    deg_parts = _sc_deg(dst_rows, ones, zeros8)

    g1, dinv = pl.pallas_call(
        _prep1_body,
        out_shape=(
            jax.ShapeDtypeStruct((NPAD, HIDDEN), _f32),
            jax.ShapeDtypeStruct((NPAD,), _f32),
        ),
    )(xp, W1, deg_parts)

    a1_parts = _sc_edge(g1, src_rows, dst_rows, zeros64, HIDDEN)

    g2 = pl.pallas_call(
        _mid_body,
        out_shape=jax.ShapeDtypeStruct((NPAD, OUT_CH), _f32),
    )(a1_parts, g1, dinv, b1, gamma, beta, W2)

    a2_parts = _sc_edge(g2, src_rows, dst_rows, zeros32, OUT_CH)

    z = pl.pallas_call(
        _final_body,
        out_shape=jax.ShapeDtypeStruct((N, OUT_CH), _f32),
    )(a2_parts, g2, dinv, b2)
    return z


# speedup vs baseline: 45.2414x; 1.0112x over previous
import jax
import jax.numpy as jnp
from jax import lax
from jax.experimental import pallas as pl
from jax.experimental.pallas import tpu as pltpu
from jax.experimental.pallas import tpu_sc as plsc

N = 10000
E = 320000
IN_CH = 128
HIDDEN = 64
OUT_CH = 32
BN_EPS = 1e-5

NC = 2
NS = 16
NW = NC * NS

EB = 125
EROWS = E // EB
RW = EROWS // NW
NPAD = 10112
RT = NPAD // NS

_f32 = jnp.float32


def _sc_mesh():
    return plsc.VectorSubcoreMesh(
        core_axis_name="c", subcore_axis_name="s", num_cores=NC, num_subcores=NS
    )


_SC_PARAMS = pltpu.CompilerParams(use_tc_tiling_on_sc=False)


NBD = 8


def _deg_body(dst_hbm, ones_hbm, zeros_hbm, out_hbm, acc_sh, dstbuf, onesbuf, *ssem):
    c = lax.axis_index("c")
    s = lax.axis_index("s")
    wid = s * NC + c
    r0 = s * RT
    pltpu.sync_copy(zeros_hbm.at[pl.ds(r0, RT)], acc_sh.at[pl.ds(r0, RT)])
    pltpu.sync_copy(ones_hbm, onesbuf)
    pltpu.sync_copy(dst_hbm.at[pl.ds(wid * RW, RW)], dstbuf)
    plsc.subcore_barrier()

    def body(i, _):
        for b in range(NBD):
            @pl.when(i > 0)
            def _wait_prev():
                pltpu.make_async_copy(onesbuf, acc_sh.at[dstbuf.at[0]], ssem[b]).wait()

            pltpu.async_copy(onesbuf, acc_sh.at[dstbuf.at[i * NBD + b]], ssem[b], add=True)
        return 0

    lax.fori_loop(0, RW // NBD, body, 0)
    for b in range(NBD):
        pltpu.make_async_copy(onesbuf, acc_sh.at[dstbuf.at[0]], ssem[b]).wait()
    plsc.subcore_barrier()
    pltpu.sync_copy(acc_sh.at[pl.ds(r0, RT)], out_hbm.at[pl.ds(c * NPAD + r0, RT)])


NB = 4
NBLK2 = RW // (2 * NB)


def _edge_body(g_hbm, src_hbm, dst_hbm, zeros_hbm, out_hbm,
               acc_sh, srcbuf, dstbuf, *bufs_and_sems):
    rows = bufs_and_sems[0:2 * NB]
    gsem = bufs_and_sems[2 * NB:3 * NB]
    ssem = bufs_and_sems[3 * NB:5 * NB]
    c = lax.axis_index("c")
    s = lax.axis_index("s")
    wid = s * NC + c
    r0 = s * RT
    pltpu.sync_copy(zeros_hbm.at[pl.ds(r0, RT)], acc_sh.at[pl.ds(r0, RT)])
    pltpu.sync_copy(src_hbm.at[pl.ds(wid * RW, RW)], srcbuf)
    pltpu.sync_copy(dst_hbm.at[pl.ds(wid * RW, RW)], dstbuf)
    plsc.subcore_barrier()

    def body(i, _):
        for half in range(2):
            j0 = (2 * i + half) * NB
            hb = rows[half * NB:(half + 1) * NB]
            hs = ssem[half * NB:(half + 1) * NB]

            @pl.when(i > 0)
            def _wait_prev():
                for b in range(NB):
                    pltpu.make_async_copy(hb[b], acc_sh.at[dstbuf.at[0]], hs[b]).wait()

            gd = [pltpu.async_copy(g_hbm.at[srcbuf.at[j0 + b]], hb[b], gsem[b])
                  for b in range(NB)]
            for b in range(NB):
                gd[b].wait()
                pltpu.async_copy(hb[b], acc_sh.at[dstbuf.at[j0 + b]], hs[b], add=True)
        return 0

    lax.fori_loop(0, NBLK2, body, 0)
    for b in range(2 * NB):
        pltpu.make_async_copy(rows[b], acc_sh.at[dstbuf.at[0]], ssem[b]).wait()
    plsc.subcore_barrier()
    pltpu.sync_copy(acc_sh.at[pl.ds(r0, RT)], out_hbm.at[pl.ds(c * NPAD + r0, RT)])


def _sc_deg(dst_rows, ones, zeros8):
    return pl.kernel(
        _deg_body,
        out_type=jax.ShapeDtypeStruct((NC * NPAD, 8), _f32),
        mesh=_sc_mesh(),
        scratch_types=(
            [
                pltpu.VMEM_SHARED((NPAD, 8), _f32),
                pltpu.VMEM((RW, EB), jnp.int32),
                pltpu.VMEM((EB, 8), _f32),
            ]
            + [pltpu.SemaphoreType.DMA] * NBD
        ),
        compiler_params=_SC_PARAMS,
    )(dst_rows, ones, zeros8)


def _sc_edge(g, src_rows, dst_rows, zerosF, F):
    return pl.kernel(
        _edge_body,
        out_type=jax.ShapeDtypeStruct((NC * NPAD, F), _f32),
        mesh=_sc_mesh(),
        scratch_types=(
            [
                pltpu.VMEM_SHARED((NPAD, F), _f32),
                pltpu.VMEM((RW, EB), jnp.int32),
                pltpu.VMEM((RW, EB), jnp.int32),
            ]
            + [pltpu.VMEM((EB, F), _f32)] * (2 * NB)
            + [pltpu.SemaphoreType.DMA] * (3 * NB)
        ),
        compiler_params=_SC_PARAMS,
    )(g, src_rows, dst_rows, zerosF)


def _h1_body(x_ref, w1_ref, h1_ref):
    h1_ref[...] = jnp.dot(x_ref[...], w1_ref[...], preferred_element_type=_f32)


def _prep1_body(h1_ref, degp_ref, g1_ref, dinv_ref):
    deg = degp_ref[0:NPAD, 0] + degp_ref[NPAD:2 * NPAD, 0] + 1.0
    dinv = lax.rsqrt(deg)
    g1_ref[0:N, :] = h1_ref[...] * dinv[0:N][:, None]
    g1_ref[N:NPAD, :] = jnp.zeros((NPAD - N, HIDDEN), _f32)
    dinv_ref[...] = dinv


def _mid_body(a1p_ref, g1_ref, dinv_ref, b1_ref, gamma_ref, beta_ref, w2_ref, g2_ref):
    dinv = dinv_ref[...]
    a1 = a1p_ref[0:NPAD, :] + a1p_ref[NPAD:2 * NPAD, :]
    pre = dinv[:, None] * (a1 + g1_ref[...]) + b1_ref[...][None, :]
    live = pre[0:N, :]
    mean = jnp.sum(live, axis=0) * (1.0 / N)
    var = jnp.sum((live - mean[None, :]) ** 2, axis=0) * (1.0 / N)
    hb = (pre - mean[None, :]) * lax.rsqrt(var + BN_EPS)[None, :]
    hb = hb * gamma_ref[...][None, :] + beta_ref[...][None, :]
    r = jnp.maximum(hb, 0.0)
    h2 = jnp.dot(r, w2_ref[...], preferred_element_type=_f32)
    g2_ref[...] = h2 * dinv[:, None]


def _final_body(a2p_ref, g2_ref, dinv_ref, b2_ref, z_ref):
    a2 = a2p_ref[0:NPAD, :] + a2p_ref[NPAD:2 * NPAD, :]
    z = dinv_ref[...][:, None] * (a2 + g2_ref[...]) + b2_ref[...][None, :]
    z_ref[...] = z[0:N, :]


def kernel(x, edge_index, W1, b1, gamma, beta, W2, b2):
    src_rows = edge_index[0].astype(jnp.int32).reshape(EROWS, EB)
    dst_rows = edge_index[1].astype(jnp.int32).reshape(EROWS, EB)
    ones = jnp.ones((EB, 8), _f32)
    zeros8 = jnp.zeros((NPAD, 8), _f32)
    zeros64 = jnp.zeros((NPAD, HIDDEN), _f32)
    zeros32 = jnp.zeros((NPAD, OUT_CH), _f32)

    h1 = pl.pallas_call(
        _h1_body,
        out_shape=jax.ShapeDtypeStruct((N, HIDDEN), _f32),
    )(x, W1)

    deg_parts = _sc_deg(dst_rows, ones, zeros8)

    g1, dinv = pl.pallas_call(
        _prep1_body,
        out_shape=(
            jax.ShapeDtypeStruct((NPAD, HIDDEN), _f32),
            jax.ShapeDtypeStruct((NPAD,), _f32),
        ),
    )(h1, deg_parts)

    a1_parts = _sc_edge(g1, src_rows, dst_rows, zeros64, HIDDEN)

    g2 = pl.pallas_call(
        _mid_body,
        out_shape=jax.ShapeDtypeStruct((NPAD, OUT_CH), _f32),
    )(a1_parts, g1, dinv, b1, gamma, beta, W2)

    a2_parts = _sc_edge(g2, src_rows, dst_rows, zeros32, OUT_CH)

    z = pl.pallas_call(
        _final_body,
        out_shape=jax.ShapeDtypeStruct((N, OUT_CH), _f32),
    )(a2_parts, g2, dinv, b2)
    return z


# speedup vs baseline: 50.6345x; 1.1192x over previous
import jax
import jax.numpy as jnp
from jax import lax
from jax.experimental import pallas as pl
from jax.experimental.pallas import tpu as pltpu
from jax.experimental.pallas import tpu_sc as plsc

N = 10000
E = 320000
IN_CH = 128
HIDDEN = 64
OUT_CH = 32
BN_EPS = 1e-5

NC = 2
NS = 16
NW = NC * NS

EB = 125
EROWS = E // EB
RW = EROWS // NW
NPAD = 10112
RT = NPAD // NS

_f32 = jnp.float32


def _sc_mesh():
    return plsc.VectorSubcoreMesh(
        core_axis_name="c", subcore_axis_name="s", num_cores=NC, num_subcores=NS
    )


_SC_PARAMS = pltpu.CompilerParams(use_tc_tiling_on_sc=False)


NBD = 8


def _deg_body(dst_hbm, ones_hbm, zeros_hbm, out_hbm, acc_sh, dstbuf, onesbuf, *ssem):
    c = lax.axis_index("c")
    s = lax.axis_index("s")
    wid = s * NC + c
    r0 = s * RT
    pltpu.sync_copy(zeros_hbm.at[pl.ds(r0, RT)], acc_sh.at[pl.ds(r0, RT)])
    pltpu.sync_copy(ones_hbm, onesbuf)
    pltpu.sync_copy(dst_hbm.at[pl.ds(wid * RW, RW)], dstbuf)
    plsc.subcore_barrier()

    def body(i, _):
        for b in range(NBD):
            @pl.when(i > 0)
            def _wait_prev():
                pltpu.make_async_copy(onesbuf, acc_sh.at[dstbuf.at[0]], ssem[b]).wait()

            pltpu.async_copy(onesbuf, acc_sh.at[dstbuf.at[i * NBD + b]], ssem[b], add=True)
        return 0

    lax.fori_loop(0, RW // NBD, body, 0)
    for b in range(NBD):
        pltpu.make_async_copy(onesbuf, acc_sh.at[dstbuf.at[0]], ssem[b]).wait()
    plsc.subcore_barrier()
    pltpu.sync_copy(acc_sh.at[pl.ds(r0, RT)],
                    out_hbm.at[pl.ds(r0, RT), pl.ds(c * 64, 8)])


NB = 4
NBLK2 = RW // (2 * NB)


def _edge_body(g_hbm, src_hbm, dst_hbm, zeros_hbm, out_hbm,
               acc_sh, srcbuf, dstbuf, *bufs_and_sems):
    rows = bufs_and_sems[0:2 * NB]
    gsem = bufs_and_sems[2 * NB:3 * NB]
    ssem = bufs_and_sems[3 * NB:5 * NB]
    c = lax.axis_index("c")
    s = lax.axis_index("s")
    wid = s * NC + c
    r0 = s * RT
    pltpu.sync_copy(zeros_hbm.at[pl.ds(r0, RT)], acc_sh.at[pl.ds(r0, RT)])
    pltpu.sync_copy(src_hbm.at[pl.ds(wid * RW, RW)], srcbuf)
    pltpu.sync_copy(dst_hbm.at[pl.ds(wid * RW, RW)], dstbuf)
    plsc.subcore_barrier()

    def body(i, _):
        for half in range(2):
            j0 = (2 * i + half) * NB
            hb = rows[half * NB:(half + 1) * NB]
            hs = ssem[half * NB:(half + 1) * NB]

            @pl.when(i > 0)
            def _wait_prev():
                for b in range(NB):
                    pltpu.make_async_copy(hb[b], acc_sh.at[dstbuf.at[0]], hs[b]).wait()

            gd = [pltpu.async_copy(g_hbm.at[srcbuf.at[j0 + b]], hb[b], gsem[b])
                  for b in range(NB)]
            for b in range(NB):
                gd[b].wait()
                pltpu.async_copy(hb[b], acc_sh.at[dstbuf.at[j0 + b]], hs[b], add=True)
        return 0

    lax.fori_loop(0, NBLK2, body, 0)
    for b in range(2 * NB):
        pltpu.make_async_copy(rows[b], acc_sh.at[dstbuf.at[0]], ssem[b]).wait()
    plsc.subcore_barrier()
    F = rows[0].shape[1]
    pltpu.sync_copy(acc_sh.at[pl.ds(r0, RT)],
                    out_hbm.at[pl.ds(r0, RT), pl.ds(c * F, F)])


def _sc_deg(dst_rows, ones, zeros8):
    return pl.kernel(
        _deg_body,
        out_type=jax.ShapeDtypeStruct((NPAD, 128), _f32),
        mesh=_sc_mesh(),
        scratch_types=(
            [
                pltpu.VMEM_SHARED((NPAD, 8), _f32),
                pltpu.VMEM((RW, EB), jnp.int32),
                pltpu.VMEM((EB, 8), _f32),
            ]
            + [pltpu.SemaphoreType.DMA] * NBD
        ),
        compiler_params=_SC_PARAMS,
    )(dst_rows, ones, zeros8)


def _sc_edge(g, src_rows, dst_rows, zerosF, F):
    return pl.kernel(
        _edge_body,
        out_type=jax.ShapeDtypeStruct((NPAD, 128), _f32),
        mesh=_sc_mesh(),
        scratch_types=(
            [
                pltpu.VMEM_SHARED((NPAD, F), _f32),
                pltpu.VMEM((RW, EB), jnp.int32),
                pltpu.VMEM((RW, EB), jnp.int32),
            ]
            + [pltpu.VMEM((EB, F), _f32)] * (2 * NB)
            + [pltpu.SemaphoreType.DMA] * (3 * NB)
        ),
        compiler_params=_SC_PARAMS,
    )(g, src_rows, dst_rows, zerosF)


def _h1_body(x_ref, w1_ref, h1_ref):
    h1_ref[...] = jnp.dot(x_ref[...], w1_ref[...], preferred_element_type=_f32)


def _prep1_body(h1_ref, degp_ref, g1_ref, dinv_ref):
    deg = degp_ref[:, 0] + degp_ref[:, 64] + 1.0
    dinv = lax.rsqrt(deg)
    g1_ref[0:N, :] = h1_ref[...] * dinv[0:N][:, None]
    g1_ref[N:NPAD, :] = jnp.zeros((NPAD - N, HIDDEN), _f32)
    dinv_ref[...] = dinv


def _mid_body(a1p_ref, g1_ref, dinv_ref, b1_ref, gamma_ref, beta_ref, w2_ref, g2_ref):
    dinv = dinv_ref[...]
    a1 = a1p_ref[:, 0:HIDDEN] + a1p_ref[:, HIDDEN:2 * HIDDEN]
    pre = dinv[:, None] * (a1 + g1_ref[...]) + b1_ref[...][None, :]
    live = pre[0:N, :]
    mean = jnp.sum(live, axis=0) * (1.0 / N)
    var = jnp.sum((live - mean[None, :]) ** 2, axis=0) * (1.0 / N)
    hb = (pre - mean[None, :]) * lax.rsqrt(var + BN_EPS)[None, :]
    hb = hb * gamma_ref[...][None, :] + beta_ref[...][None, :]
    r = jnp.maximum(hb, 0.0)
    h2 = jnp.dot(r, w2_ref[...], preferred_element_type=_f32)
    g2_ref[...] = h2 * dinv[:, None]


def _final_body(a2p_ref, g2_ref, dinv_ref, b2_ref, z_ref):
    a2 = a2p_ref[:, 0:OUT_CH] + a2p_ref[:, OUT_CH:2 * OUT_CH]
    z = dinv_ref[...][:, None] * (a2 + g2_ref[...]) + b2_ref[...][None, :]
    z_ref[...] = z[0:N, :]


def kernel(x, edge_index, W1, b1, gamma, beta, W2, b2):
    src_rows = edge_index[0].astype(jnp.int32).reshape(EROWS, EB)
    dst_rows = edge_index[1].astype(jnp.int32).reshape(EROWS, EB)
    ones = jnp.ones((EB, 8), _f32)
    zeros8 = jnp.zeros((NPAD, 8), _f32)
    zeros64 = jnp.zeros((NPAD, HIDDEN), _f32)
    zeros32 = jnp.zeros((NPAD, OUT_CH), _f32)

    h1 = pl.pallas_call(
        _h1_body,
        out_shape=jax.ShapeDtypeStruct((N, HIDDEN), _f32),
    )(x, W1)

    deg_parts = _sc_deg(dst_rows, ones, zeros8)

    g1, dinv = pl.pallas_call(
        _prep1_body,
        out_shape=(
            jax.ShapeDtypeStruct((NPAD, HIDDEN), _f32),
            jax.ShapeDtypeStruct((NPAD,), _f32),
        ),
    )(h1, deg_parts)

    a1_parts = _sc_edge(g1, src_rows, dst_rows, zeros64, HIDDEN)

    g2 = pl.pallas_call(
        _mid_body,
        out_shape=jax.ShapeDtypeStruct((NPAD, OUT_CH), _f32),
    )(a1_parts, g1, dinv, b1, gamma, beta, W2)

    a2_parts = _sc_edge(g2, src_rows, dst_rows, zeros32, OUT_CH)

    z = pl.pallas_call(
        _final_body,
        out_shape=jax.ShapeDtypeStruct((N, OUT_CH), _f32),
    )(a2_parts, g2, dinv, b2)
    return z
